# baseline (speedup 1.0000x reference)
import dataclasses
import functools

import jax
import jax.numpy as jnp
from jax import lax
from jax.experimental import pallas as pl
from jax.experimental.pallas import tpu as pltpu
from jax.experimental.pallas import tpu_sc as plsc

N = 10000
E = 320000
D = 128
H = 64
C = 2

NC = 2
NS = 16
NW = NC * NS
B = 128
NP = 10240
RPT = NP // NS
EWP = 10240
NSUB = EWP // B
EP = NW * EWP

_f32 = jnp.float32


def _tc_prologue_body(x_ref, w_ref, asv_ref, adv_ref,
                      h_out, as_out, ad_out, mp_out):
    h = jnp.dot(x_ref[...], w_ref[...], preferred_element_type=_f32)
    h_out[...] = h
    a_s = jnp.sum(h * asv_ref[...], axis=1, keepdims=True)
    a_d = jnp.sum(h * adv_ref[...], axis=1, keepdims=True)
    as_out[...] = a_s
    ad_out[...] = a_d
    z = jnp.max(a_s) + a_d
    mp_out[...] = jnp.where(z > 0, z, 0.2 * z)


def _tc_mid_body(accp_ref, denp_ref, b1_ref, w2_ref, asv_ref, adv_ref,
                 as_out, ad_out, mp_out, hc0_out, hc1_out):
    acc = accp_ref[0] + accp_ref[1]
    den = denp_ref[0] + denp_ref[1]
    hr = jnp.maximum(acc / (den + 1e-16) + b1_ref[...], 0.0)
    h2 = jnp.dot(hr, w2_ref[...], preferred_element_type=_f32)
    a_s = jnp.sum(h2 * asv_ref[...], axis=1, keepdims=True)
    a_d = jnp.sum(h2 * adv_ref[...], axis=1, keepdims=True)
    as_out[...] = a_s
    ad_out[...] = a_d
    z = jnp.max(a_s) + a_d
    mp_out[...] = jnp.where(z > 0, z, 0.2 * z)
    hc0_out[...] = h2[:, 0:1]
    hc1_out[...] = h2[:, 1:2]


def _tc_epilogue_body(n0_ref, n1_ref, d2_ref, b2_ref, out_ref):
    d = d2_ref[0] + d2_ref[1] + 1e-16
    o0 = (n0_ref[0] + n0_ref[1]) / d
    o1 = (n1_ref[0] + n1_ref[1]) / d
    out_ref[...] = jnp.concatenate([o0, o1], axis=1) + b2_ref[...]


def _leaky(e):
    return jnp.where(e > 0, e, 0.2 * e)


def _sc_layer1_body(src_hbm, dst_hbm, as_hbm, ad_hbm, mp_hbm, h_hbm,
                    acc_out, den_out,
                    as_t, ad_t, mp_t, sv, dv, wv, rows0, rows1, zrow, zden,
                    acc_sh, den_sh, sem0, sem1):
    cid = lax.axis_index("c")
    sid = lax.axis_index("s")
    wid = cid * NS + sid

    pltpu.sync_copy(as_hbm, as_t)
    pltpu.sync_copy(ad_hbm, ad_t)
    pltpu.sync_copy(mp_hbm, mp_t)

    zeros16 = jnp.zeros((16,), _f32)

    @pl.loop(0, 16)
    def _(r):
        @pl.loop(0, H // 16)
        def _(c4):
            zrow[r, pl.ds(c4 * 16, 16)] = zeros16

    @pl.loop(0, RPT // 16)
    def _(i):
        pltpu.sync_copy(zrow, acc_sh.at[pl.ds(sid * RPT + i * 16, 16)])

    @pl.loop(0, RPT // 16)
    def _(i):
        zden[pl.ds(i * 16, 16)] = zeros16
    pltpu.sync_copy(zden, den_sh.at[pl.ds(sid * RPT, RPT)])
    plsc.subcore_barrier()

    pltpu.sync_copy(src_hbm.at[wid], sv)
    pltpu.sync_copy(dst_hbm.at[wid], dv)

    pltpu.async_copy(h_hbm.at[sv.at[0]], rows0, sem0)
    pltpu.async_copy(h_hbm.at[sv.at[1]], rows1, sem1)

    def _process1(j, rows, sem, may_issue):
        @pl.loop(0, B // 16)
        def _(i):
            s16 = sv[j, pl.ds(i * 16, 16)]
            d16 = dv[j, pl.ds(i * 16, 16)]
            a_s = plsc.load_gather(as_t, [s16])
            a_d = plsc.load_gather(ad_t, [d16])
            mp = plsc.load_gather(mp_t, [d16])
            e = _leaky(a_s + a_d)
            wv[pl.ds(i * 16, 16)] = jnp.exp(e - mp)

        pltpu.make_async_copy(h_hbm.at[sv.at[j]], rows, sem).wait()

        @pl.loop(0, B // 16)
        def _(i):
            w16 = wv[pl.ds(i * 16, 16)]
            for l in range(16):
                w = w16[l]
                for c4 in range(H // 16):
                    r = i * 16 + l
                    rows[r, pl.ds(c4 * 16, 16)] = rows[r, pl.ds(c4 * 16, 16)] * w

        pltpu.sync_copy(rows, acc_sh.at[dv.at[j]], add=True)
        pltpu.sync_copy(wv, den_sh.at[dv.at[j]], add=True)
        if may_issue is not None:
            @pl.when(may_issue)
            def _():
                pltpu.async_copy(h_hbm.at[sv.at[j + 2]], rows, sem)

    @pl.loop(0, NSUB // 2)
    def _(k):
        _process1(2 * k, rows0, sem0, 2 * k + 2 < NSUB)
        _process1(2 * k + 1, rows1, sem1, 2 * k + 3 < NSUB)

    plsc.subcore_barrier()
    pltpu.sync_copy(acc_sh.at[pl.ds(sid * RPT, RPT)],
                    acc_out.at[cid].at[pl.ds(sid * RPT, RPT)])
    pltpu.sync_copy(den_sh.at[pl.ds(sid * RPT, RPT)],
                    den_out.at[cid].at[pl.ds(sid * RPT, RPT)])


def _sc_layer2_body(src_hbm, dst_hbm, as_hbm, ad_hbm, mp_hbm, h0_hbm, h1_hbm,
                    n0_out, n1_out, den_out,
                    as_t, ad_t, mp_t, h0_t, h1_t, sv, dv,
                    wv0, p0v0, p1v0, wv1, p0v1, p1v1, zden,
                    n0_sh, n1_sh, den_sh, sem0, sem1):
    cid = lax.axis_index("c")
    sid = lax.axis_index("s")
    wid = cid * NS + sid

    pltpu.sync_copy(as_hbm, as_t)
    pltpu.sync_copy(ad_hbm, ad_t)
    pltpu.sync_copy(mp_hbm, mp_t)
    pltpu.sync_copy(h0_hbm, h0_t)
    pltpu.sync_copy(h1_hbm, h1_t)

    zeros16 = jnp.zeros((16,), _f32)

    @pl.loop(0, RPT // 16)
    def _(i):
        zden[pl.ds(i * 16, 16)] = zeros16
    pltpu.sync_copy(zden, n0_sh.at[pl.ds(sid * RPT, RPT)])
    pltpu.sync_copy(zden, n1_sh.at[pl.ds(sid * RPT, RPT)])
    pltpu.sync_copy(zden, den_sh.at[pl.ds(sid * RPT, RPT)])
    plsc.subcore_barrier()

    pltpu.sync_copy(src_hbm.at[wid], sv)
    pltpu.sync_copy(dst_hbm.at[wid], dv)

    def _process2(j, wv, p0v, p1v, sem, do_wait):
        def _wait_prev():
            pltpu.make_async_copy(p0v, n0_sh.at[dv.at[j]], sem).wait()
            pltpu.make_async_copy(p1v, n1_sh.at[dv.at[j]], sem).wait()
            pltpu.make_async_copy(wv, den_sh.at[dv.at[j]], sem).wait()
        if do_wait == "always":
            _wait_prev()
        elif do_wait is not None:
            @pl.when(do_wait)
            def _():
                _wait_prev()

        @pl.loop(0, B // 16)
        def _(i):
            s16 = sv[j, pl.ds(i * 16, 16)]
            d16 = dv[j, pl.ds(i * 16, 16)]
            a_s = plsc.load_gather(as_t, [s16])
            a_d = plsc.load_gather(ad_t, [d16])
            mp = plsc.load_gather(mp_t, [d16])
            h0 = plsc.load_gather(h0_t, [s16])
            h1 = plsc.load_gather(h1_t, [s16])
            e = _leaky(a_s + a_d)
            w = jnp.exp(e - mp)
            sl = pl.ds(i * 16, 16)
            wv[sl] = w
            p0v[sl] = w * h0
            p1v[sl] = w * h1

        pltpu.async_copy(p0v, n0_sh.at[dv.at[j]], sem, add=True)
        pltpu.async_copy(p1v, n1_sh.at[dv.at[j]], sem, add=True)
        pltpu.async_copy(wv, den_sh.at[dv.at[j]], sem, add=True)

    @pl.loop(0, NSUB // 2)
    def _(k):
        _process2(2 * k, wv0, p0v0, p1v0, sem0, k > 0)
        _process2(2 * k + 1, wv1, p0v1, p1v1, sem1, k > 0)

    pltpu.make_async_copy(p0v0, n0_sh.at[dv.at[0]], sem0).wait()
    pltpu.make_async_copy(p1v0, n1_sh.at[dv.at[0]], sem0).wait()
    pltpu.make_async_copy(wv0, den_sh.at[dv.at[0]], sem0).wait()
    pltpu.make_async_copy(p0v1, n0_sh.at[dv.at[0]], sem1).wait()
    pltpu.make_async_copy(p1v1, n1_sh.at[dv.at[0]], sem1).wait()
    pltpu.make_async_copy(wv1, den_sh.at[dv.at[0]], sem1).wait()

    plsc.subcore_barrier()
    pltpu.sync_copy(n0_sh.at[pl.ds(sid * RPT, RPT)],
                    n0_out.at[cid].at[pl.ds(sid * RPT, RPT)])
    pltpu.sync_copy(n1_sh.at[pl.ds(sid * RPT, RPT)],
                    n1_out.at[cid].at[pl.ds(sid * RPT, RPT)])
    pltpu.sync_copy(den_sh.at[pl.ds(sid * RPT, RPT)],
                    den_out.at[cid].at[pl.ds(sid * RPT, RPT)])


def kernel(x, edge_index, W1, att_src1, att_dst1, b1, W2, att_src2, att_dst2, b2):
    npad = EP - E
    pad_src = jnp.zeros((npad,), jnp.int32)
    pad_dst = N + (jnp.arange(npad, dtype=jnp.int32) % (NP - N))
    src3d = jnp.concatenate([edge_index[0], pad_src]).reshape(NW, NSUB, B)
    dst3d = jnp.concatenate([edge_index[1], pad_dst]).reshape(NW, NSUB, B)

    h1, as1, ad1, mp1 = pl.pallas_call(
        _tc_prologue_body,
        out_shape=[
            jax.ShapeDtypeStruct((N, H), _f32),
            jax.ShapeDtypeStruct((N, 1), _f32),
            jax.ShapeDtypeStruct((N, 1), _f32),
            jax.ShapeDtypeStruct((N, 1), _f32),
        ],
    )(x, W1, att_src1.reshape(1, H), att_dst1.reshape(1, H))

    mesh = plsc.VectorSubcoreMesh(core_axis_name="c", subcore_axis_name="s")
    cp = pltpu.CompilerParams()
    for _f, _v in (("needs_layout_passes", False), ("use_tc_tiling_on_sc", False)):
        if _f in pltpu.CompilerParams.__dataclass_fields__:
            cp = dataclasses.replace(cp, **{_f: _v})

    sc1 = functools.partial(
        pl.kernel,
        out_type=[
            jax.ShapeDtypeStruct((NC, NP, H), _f32),
            jax.ShapeDtypeStruct((NC, NP), _f32),
        ],
        mesh=mesh,
        compiler_params=cp,
        scratch_types=[
            pltpu.VMEM((NP,), _f32),
            pltpu.VMEM((NP,), _f32),
            pltpu.VMEM((NP,), _f32),
            pltpu.VMEM((NSUB, B), jnp.int32),
            pltpu.VMEM((NSUB, B), jnp.int32),
            pltpu.VMEM((B,), _f32),
            pltpu.VMEM((B, H), _f32),
            pltpu.VMEM((B, H), _f32),
            pltpu.VMEM((16, H), _f32),
            pltpu.VMEM((RPT,), _f32),
            pltpu.VMEM_SHARED((NP, H), _f32),
            pltpu.VMEM_SHARED((NP,), _f32),
            pltpu.SemaphoreType.DMA,
            pltpu.SemaphoreType.DMA,
        ],
    )(_sc_layer1_body)
    zpad = jnp.zeros((NP - N,), _f32)
    accp, denp = sc1(src3d, dst3d,
                     jnp.concatenate([as1.reshape(N), zpad]),
                     jnp.concatenate([ad1.reshape(N), zpad]),
                     jnp.concatenate([mp1.reshape(N), zpad]), h1)

    as2, ad2, mp2, hc0, hc1 = pl.pallas_call(
        _tc_mid_body,
        out_shape=[
            jax.ShapeDtypeStruct((NP, 1), _f32),
            jax.ShapeDtypeStruct((NP, 1), _f32),
            jax.ShapeDtypeStruct((NP, 1), _f32),
            jax.ShapeDtypeStruct((NP, 1), _f32),
            jax.ShapeDtypeStruct((NP, 1), _f32),
        ],
    )(accp, denp.reshape(NC, NP, 1), b1.reshape(1, H), W2,
      att_src2.reshape(1, C), att_dst2.reshape(1, C))

    sc2 = functools.partial(
        pl.kernel,
        out_type=[
            jax.ShapeDtypeStruct((NC, NP), _f32),
            jax.ShapeDtypeStruct((NC, NP), _f32),
            jax.ShapeDtypeStruct((NC, NP), _f32),
        ],
        mesh=mesh,
        compiler_params=cp,
        scratch_types=[
            pltpu.VMEM((NP,), _f32),
            pltpu.VMEM((NP,), _f32),
            pltpu.VMEM((NP,), _f32),
            pltpu.VMEM((NP,), _f32),
            pltpu.VMEM((NP,), _f32),
            pltpu.VMEM((NSUB, B), jnp.int32),
            pltpu.VMEM((NSUB, B), jnp.int32),
            pltpu.VMEM((B,), _f32),
            pltpu.VMEM((B,), _f32),
            pltpu.VMEM((B,), _f32),
            pltpu.VMEM((B,), _f32),
            pltpu.VMEM((B,), _f32),
            pltpu.VMEM((B,), _f32),
            pltpu.VMEM((RPT,), _f32),
            pltpu.VMEM_SHARED((NP,), _f32),
            pltpu.VMEM_SHARED((NP,), _f32),
            pltpu.VMEM_SHARED((NP,), _f32),
            pltpu.SemaphoreType.DMA,
            pltpu.SemaphoreType.DMA,
        ],
    )(_sc_layer2_body)
    n0, n1, den2 = sc2(src3d, dst3d,
                       as2.reshape(NP), ad2.reshape(NP), mp2.reshape(NP),
                       hc0.reshape(NP), hc1.reshape(NP))

    out = pl.pallas_call(
        _tc_epilogue_body,
        out_shape=jax.ShapeDtypeStruct((NP, C), _f32),
    )(n0.reshape(NC, NP, 1), n1.reshape(NC, NP, 1), den2.reshape(NC, NP, 1),
      b2.reshape(1, C))

    return out[:N]


# speedup vs baseline: 1.1846x; 1.1846x over previous
import dataclasses
import functools

import jax
import jax.numpy as jnp
from jax import lax
from jax.experimental import pallas as pl
from jax.experimental.pallas import tpu as pltpu
from jax.experimental.pallas import tpu_sc as plsc

N = 10000
E = 320000
D = 128
H = 64
C = 2

NC = 2
NS = 16
NW = NC * NS
EW = E // NW
B = 80
NSUB = EW // B
NP = 10240
RPT = NP // NS

_f32 = jnp.float32


def _tc_prologue_body(x_ref, w_ref, asv_ref, adv_ref,
                      h_out, as_out, ad_out, mp_out):
    h = jnp.dot(x_ref[...], w_ref[...], preferred_element_type=_f32)
    h_out[...] = h
    a_s = jnp.sum(h * asv_ref[...], axis=1, keepdims=True)
    a_d = jnp.sum(h * adv_ref[...], axis=1, keepdims=True)
    as_out[...] = a_s
    ad_out[...] = a_d
    z = jnp.max(a_s) + a_d
    mp_out[...] = jnp.where(z > 0, z, 0.2 * z)


def _tc_mid_body(accp_ref, denp_ref, b1_ref, w2_ref, asv_ref, adv_ref,
                 as_out, ad_out, mp_out, hc0_out, hc1_out):
    acc = accp_ref[0] + accp_ref[1]
    den = denp_ref[0] + denp_ref[1]
    hr = jnp.maximum(acc / (den + 1e-16) + b1_ref[...], 0.0)
    h2 = jnp.dot(hr, w2_ref[...], preferred_element_type=_f32)
    a_s = jnp.sum(h2 * asv_ref[...], axis=1, keepdims=True)
    a_d = jnp.sum(h2 * adv_ref[...], axis=1, keepdims=True)
    as_out[...] = a_s
    ad_out[...] = a_d
    z = jnp.max(a_s) + a_d
    mp_out[...] = jnp.where(z > 0, z, 0.2 * z)
    hc0_out[...] = h2[:, 0:1]
    hc1_out[...] = h2[:, 1:2]


def _tc_epilogue_body(n0_ref, n1_ref, d2_ref, b2_ref, out_ref):
    d = d2_ref[0] + d2_ref[1] + 1e-16
    o0 = (n0_ref[0] + n0_ref[1]) / d
    o1 = (n1_ref[0] + n1_ref[1]) / d
    out_ref[...] = jnp.concatenate([o0, o1], axis=1) + b2_ref[...]


def _leaky(e):
    return jnp.where(e > 0, e, 0.2 * e)


def _when(cond, fn):
    if cond is None:
        return
    if cond is True:
        fn()
        return

    @pl.when(cond)
    def _():
        fn()


def _sc_layer1_body(src_hbm, dst_hbm, as_hbm, ad_hbm, mp_hbm, h_hbm,
                    acc_out, den_out,
                    as_t, ad_t, mp_t, sv, dv,
                    wv0, wv1, wv2, wv3, rb0, rb1, rb2, rb3, zrow, zden,
                    acc_sh, den_sh,
                    sg0, sg1, sg2, sg3, sr0, sr1, sr2, sr3,
                    sw0, sw1, sw2, sw3):
    cid = lax.axis_index("c")
    sid = lax.axis_index("s")
    wid = cid * NS + sid

    pltpu.sync_copy(as_hbm, as_t)
    pltpu.sync_copy(ad_hbm, ad_t)
    pltpu.sync_copy(mp_hbm, mp_t)

    zeros16 = jnp.zeros((16,), _f32)

    @pl.loop(0, 16)
    def _(r):
        @pl.loop(0, H // 16)
        def _(c4):
            zrow[r, pl.ds(c4 * 16, 16)] = zeros16

    @pl.loop(0, RPT // 16)
    def _(i):
        pltpu.sync_copy(zrow, acc_sh.at[pl.ds(sid * RPT + i * 16, 16)])

    @pl.loop(0, RPT // 16)
    def _(i):
        zden[pl.ds(i * 16, 16)] = zeros16
    pltpu.sync_copy(zden, den_sh.at[pl.ds(sid * RPT, RPT)])
    plsc.subcore_barrier()

    pltpu.sync_copy(src_hbm.at[wid], sv)
    pltpu.sync_copy(dst_hbm.at[wid], dv)

    ring = [(rb0, wv0, sg0, sr0, sw0), (rb1, wv1, sg1, sr1, sw1),
            (rb2, wv2, sg2, sr2, sw2), (rb3, wv3, sg3, sr3, sw3)]

    pltpu.async_copy(h_hbm.at[sv.at[0]], rb0, sg0)
    pltpu.async_copy(h_hbm.at[sv.at[1]], rb1, sg1)

    def _process1(j, q, have_old_w, have_prev_r, issue_next):
        rows, wv, sg, sr, sw = ring[q]
        qn = (q + 2) % 4
        rows_n, _, sgn, srn, _ = ring[qn]

        _when(have_old_w,
              lambda: pltpu.make_async_copy(wv, den_sh.at[dv.at[0]], sw).wait())

        @pl.loop(0, B // 16)
        def _(i):
            s16 = sv[j, pl.ds(i * 16, 16)]
            d16 = dv[j, pl.ds(i * 16, 16)]
            a_s = plsc.load_gather(as_t, [s16])
            a_d = plsc.load_gather(ad_t, [d16])
            mp = plsc.load_gather(mp_t, [d16])
            e = _leaky(a_s + a_d)
            wv[pl.ds(i * 16, 16)] = jnp.exp(e - mp)

        pltpu.make_async_copy(h_hbm.at[sv.at[j]], rows, sg).wait()

        @pl.loop(0, B // 16)
        def _(i):
            w16 = wv[pl.ds(i * 16, 16)]
            for l in range(16):
                w = w16[l]
                for c4 in range(H // 16):
                    r = i * 16 + l
                    rows[r, pl.ds(c4 * 16, 16)] = rows[r, pl.ds(c4 * 16, 16)] * w

        pltpu.async_copy(rows, acc_sh.at[dv.at[j]], sr, add=True)
        pltpu.async_copy(wv, den_sh.at[dv.at[j]], sw, add=True)

        def _issue():
            _when(have_prev_r,
                  lambda: pltpu.make_async_copy(
                      rows_n, acc_sh.at[dv.at[0]], srn).wait())
            pltpu.async_copy(h_hbm.at[sv.at[j + 2]], rows_n, sgn)
        _when(issue_next, _issue)

    @pl.loop(0, NSUB // 4)
    def _(k):
        _process1(4 * k + 0, 0, k > 0, k > 0, True)
        _process1(4 * k + 1, 1, k > 0, k > 0, True)
        _process1(4 * k + 2, 2, k > 0, True, True)
        _process1(4 * k + 3, 3, k > 0, True, k < NSUB // 4 - 1)

    _process1(NSUB - 1, 0, True, None, None)

    for q in (1, 2, 3, 0):
        rows, wv, _, sr, sw = ring[q]
        pltpu.make_async_copy(rows, acc_sh.at[dv.at[0]], sr).wait()
        pltpu.make_async_copy(wv, den_sh.at[dv.at[0]], sw).wait()

    plsc.subcore_barrier()
    pltpu.sync_copy(acc_sh.at[pl.ds(sid * RPT, RPT)],
                    acc_out.at[cid].at[pl.ds(sid * RPT, RPT)])
    pltpu.sync_copy(den_sh.at[pl.ds(sid * RPT, RPT)],
                    den_out.at[cid].at[pl.ds(sid * RPT, RPT)])


def _sc_layer2_body(src_hbm, dst_hbm, as_hbm, ad_hbm, mp_hbm, h0_hbm, h1_hbm,
                    n0_out, n1_out, den_out,
                    as_t, ad_t, mp_t, h0_t, h1_t, sv, dv,
                    wv0, p0v0, p1v0, wv1, p0v1, p1v1, zden,
                    n0_sh, n1_sh, den_sh, sem0, sem1):
    cid = lax.axis_index("c")
    sid = lax.axis_index("s")
    wid = cid * NS + sid

    pltpu.sync_copy(as_hbm, as_t)
    pltpu.sync_copy(ad_hbm, ad_t)
    pltpu.sync_copy(mp_hbm, mp_t)
    pltpu.sync_copy(h0_hbm, h0_t)
    pltpu.sync_copy(h1_hbm, h1_t)

    zeros16 = jnp.zeros((16,), _f32)

    @pl.loop(0, RPT // 16)
    def _(i):
        zden[pl.ds(i * 16, 16)] = zeros16
    pltpu.sync_copy(zden, n0_sh.at[pl.ds(sid * RPT, RPT)])
    pltpu.sync_copy(zden, n1_sh.at[pl.ds(sid * RPT, RPT)])
    pltpu.sync_copy(zden, den_sh.at[pl.ds(sid * RPT, RPT)])
    plsc.subcore_barrier()

    pltpu.sync_copy(src_hbm.at[wid], sv)
    pltpu.sync_copy(dst_hbm.at[wid], dv)

    def _process2(j, wv, p0v, p1v, sem, do_wait):
        def _wait_prev():
            pltpu.make_async_copy(p0v, n0_sh.at[dv.at[j]], sem).wait()
            pltpu.make_async_copy(p1v, n1_sh.at[dv.at[j]], sem).wait()
            pltpu.make_async_copy(wv, den_sh.at[dv.at[j]], sem).wait()
        _when(do_wait, _wait_prev)

        @pl.loop(0, B // 16)
        def _(i):
            s16 = sv[j, pl.ds(i * 16, 16)]
            d16 = dv[j, pl.ds(i * 16, 16)]
            a_s = plsc.load_gather(as_t, [s16])
            a_d = plsc.load_gather(ad_t, [d16])
            mp = plsc.load_gather(mp_t, [d16])
            h0 = plsc.load_gather(h0_t, [s16])
            h1 = plsc.load_gather(h1_t, [s16])
            e = _leaky(a_s + a_d)
            w = jnp.exp(e - mp)
            sl = pl.ds(i * 16, 16)
            wv[sl] = w
            p0v[sl] = w * h0
            p1v[sl] = w * h1

        pltpu.async_copy(p0v, n0_sh.at[dv.at[j]], sem, add=True)
        pltpu.async_copy(p1v, n1_sh.at[dv.at[j]], sem, add=True)
        pltpu.async_copy(wv, den_sh.at[dv.at[j]], sem, add=True)

    @pl.loop(0, NSUB // 2)
    def _(k):
        _process2(2 * k, wv0, p0v0, p1v0, sem0, k > 0)
        _process2(2 * k + 1, wv1, p0v1, p1v1, sem1, k > 0)

    _process2(NSUB - 1, wv0, p0v0, p1v0, sem0, True)
    pltpu.make_async_copy(p0v0, n0_sh.at[dv.at[0]], sem0).wait()
    pltpu.make_async_copy(p1v0, n1_sh.at[dv.at[0]], sem0).wait()
    pltpu.make_async_copy(wv0, den_sh.at[dv.at[0]], sem0).wait()
    pltpu.make_async_copy(p0v1, n0_sh.at[dv.at[0]], sem1).wait()
    pltpu.make_async_copy(p1v1, n1_sh.at[dv.at[0]], sem1).wait()
    pltpu.make_async_copy(wv1, den_sh.at[dv.at[0]], sem1).wait()

    plsc.subcore_barrier()
    pltpu.sync_copy(n0_sh.at[pl.ds(sid * RPT, RPT)],
                    n0_out.at[cid].at[pl.ds(sid * RPT, RPT)])
    pltpu.sync_copy(n1_sh.at[pl.ds(sid * RPT, RPT)],
                    n1_out.at[cid].at[pl.ds(sid * RPT, RPT)])
    pltpu.sync_copy(den_sh.at[pl.ds(sid * RPT, RPT)],
                    den_out.at[cid].at[pl.ds(sid * RPT, RPT)])


def kernel(x, edge_index, W1, att_src1, att_dst1, b1, W2, att_src2, att_dst2, b2):
    src3d = edge_index[0].reshape(NW, NSUB, B)
    dst3d = edge_index[1].reshape(NW, NSUB, B)

    h1, as1, ad1, mp1 = pl.pallas_call(
        _tc_prologue_body,
        out_shape=[
            jax.ShapeDtypeStruct((N, H), _f32),
            jax.ShapeDtypeStruct((N, 1), _f32),
            jax.ShapeDtypeStruct((N, 1), _f32),
            jax.ShapeDtypeStruct((N, 1), _f32),
        ],
    )(x, W1, att_src1.reshape(1, H), att_dst1.reshape(1, H))

    mesh = plsc.VectorSubcoreMesh(core_axis_name="c", subcore_axis_name="s")
    cp = pltpu.CompilerParams()
    for _f, _v in (("needs_layout_passes", False), ("use_tc_tiling_on_sc", False)):
        if _f in pltpu.CompilerParams.__dataclass_fields__:
            cp = dataclasses.replace(cp, **{_f: _v})

    sc1 = functools.partial(
        pl.kernel,
        out_type=[
            jax.ShapeDtypeStruct((NC, NP, H), _f32),
            jax.ShapeDtypeStruct((NC, NP), _f32),
        ],
        mesh=mesh,
        compiler_params=cp,
        scratch_types=(
            [pltpu.VMEM((N,), _f32)] * 3 +
            [pltpu.VMEM((NSUB, B), jnp.int32)] * 2 +
            [pltpu.VMEM((B,), _f32)] * 4 +
            [pltpu.VMEM((B, H), _f32)] * 4 +
            [pltpu.VMEM((16, H), _f32),
             pltpu.VMEM((RPT,), _f32),
             pltpu.VMEM_SHARED((NP, H), _f32),
             pltpu.VMEM_SHARED((NP,), _f32)] +
            [pltpu.SemaphoreType.DMA] * 12
        ),
    )(_sc_layer1_body)
    accp, denp = sc1(src3d, dst3d,
                     as1.reshape(N), ad1.reshape(N), mp1.reshape(N), h1)

    as2, ad2, mp2, hc0, hc1 = pl.pallas_call(
        _tc_mid_body,
        out_shape=[
            jax.ShapeDtypeStruct((NP, 1), _f32),
            jax.ShapeDtypeStruct((NP, 1), _f32),
            jax.ShapeDtypeStruct((NP, 1), _f32),
            jax.ShapeDtypeStruct((NP, 1), _f32),
            jax.ShapeDtypeStruct((NP, 1), _f32),
        ],
    )(accp, denp.reshape(NC, NP, 1), b1.reshape(1, H), W2,
      att_src2.reshape(1, C), att_dst2.reshape(1, C))

    sc2 = functools.partial(
        pl.kernel,
        out_type=[
            jax.ShapeDtypeStruct((NC, NP), _f32),
            jax.ShapeDtypeStruct((NC, NP), _f32),
            jax.ShapeDtypeStruct((NC, NP), _f32),
        ],
        mesh=mesh,
        compiler_params=cp,
        scratch_types=(
            [pltpu.VMEM((NP,), _f32)] * 5 +
            [pltpu.VMEM((NSUB, B), jnp.int32)] * 2 +
            [pltpu.VMEM((B,), _f32)] * 6 +
            [pltpu.VMEM((RPT,), _f32),
             pltpu.VMEM_SHARED((NP,), _f32),
             pltpu.VMEM_SHARED((NP,), _f32),
             pltpu.VMEM_SHARED((NP,), _f32)] +
            [pltpu.SemaphoreType.DMA] * 2
        ),
    )(_sc_layer2_body)
    n0, n1, den2 = sc2(src3d, dst3d,
                       as2.reshape(NP), ad2.reshape(NP), mp2.reshape(NP),
                       hc0.reshape(NP), hc1.reshape(NP))

    out = pl.pallas_call(
        _tc_epilogue_body,
        out_shape=jax.ShapeDtypeStruct((NP, C), _f32),
    )(n0.reshape(NC, NP, 1), n1.reshape(NC, NP, 1), den2.reshape(NC, NP, 1),
      b2.reshape(1, C))

    return out[:N]


# speedup vs baseline: 1.2101x; 1.0215x over previous
import dataclasses
import functools

import jax
import jax.numpy as jnp
from jax import lax
from jax.experimental import pallas as pl
from jax.experimental.pallas import tpu as pltpu
from jax.experimental.pallas import tpu_sc as plsc

N = 10000
E = 320000
D = 128
H = 64
C = 2

NC = 2
NS = 16
NW = NC * NS
EW = E // NW
B = 80
NSUB = EW // B
NP = 10240
RPT = NP // NS

_f32 = jnp.float32


def _tc_prologue_body(x_ref, w_ref, asv_ref, adv_ref,
                      h_out, as_out, ad_out, mp_out):
    h = jnp.dot(x_ref[...], w_ref[...], preferred_element_type=_f32)
    h_out[...] = h
    a_s = jnp.sum(h * asv_ref[...], axis=1, keepdims=True)
    a_d = jnp.sum(h * adv_ref[...], axis=1, keepdims=True)
    as_out[...] = a_s
    ad_out[...] = a_d
    z = jnp.max(a_s) + a_d
    mp_out[...] = jnp.where(z > 0, z, 0.2 * z)


def _tc_mid_body(accp_ref, denp_ref, b1_ref, w2_ref, asv_ref, adv_ref, t_out):
    acc = accp_ref[0] + accp_ref[1]
    den = denp_ref[0] + denp_ref[1]
    hr = jnp.maximum(acc / (den + 1e-16) + b1_ref[...], 0.0)
    h2 = jnp.dot(hr, w2_ref[...], preferred_element_type=_f32)
    a_s = jnp.sum(h2 * asv_ref[...], axis=1, keepdims=True)
    a_d = jnp.sum(h2 * adv_ref[...], axis=1, keepdims=True)
    z = jnp.max(a_s) + a_d
    mp = jnp.where(z > 0, z, 0.2 * z)
    zero = jnp.zeros_like(a_s)
    t_out[...] = jnp.concatenate(
        [a_s, a_d, mp, h2[:, 0:1], h2[:, 1:2], zero, zero, zero], axis=1)


def _tc_epilogue_body(n0_ref, n1_ref, d2_ref, b2_ref, out_ref):
    d = d2_ref[0] + d2_ref[1] + 1e-16
    o0 = (n0_ref[0] + n0_ref[1]) / d
    o1 = (n1_ref[0] + n1_ref[1]) / d
    out_ref[...] = jnp.concatenate([o0, o1], axis=1) + b2_ref[...]


def _leaky(e):
    return jnp.where(e > 0, e, 0.2 * e)


def _when(cond, fn):
    if cond is None:
        return
    if cond is True:
        fn()
        return

    @pl.when(cond)
    def _():
        fn()


def _sc_layer1_body(src_hbm, dst_hbm, as_hbm, ad_hbm, mp_hbm, h_hbm,
                    acc_out, den_out,
                    as_t, ad_t, mp_t, sv, dv,
                    wv0, wv1, wv2, wv3, rb0, rb1, rb2, rb3, zrow, zden,
                    acc_sh, den_sh,
                    sg0, sg1, sg2, sg3, sr0, sr1, sr2, sr3,
                    sw0, sw1, sw2, sw3):
    cid = lax.axis_index("c")
    sid = lax.axis_index("s")
    wid = cid * NS + sid

    pltpu.sync_copy(as_hbm, as_t)
    pltpu.sync_copy(ad_hbm, ad_t)
    pltpu.sync_copy(mp_hbm, mp_t)

    zeros16 = jnp.zeros((16,), _f32)

    @pl.loop(0, 16)
    def _(r):
        @pl.loop(0, H // 16)
        def _(c4):
            zrow[r, pl.ds(c4 * 16, 16)] = zeros16

    @pl.loop(0, RPT // 16)
    def _(i):
        pltpu.sync_copy(zrow, acc_sh.at[pl.ds(sid * RPT + i * 16, 16)])

    @pl.loop(0, RPT // 16)
    def _(i):
        zden[pl.ds(i * 16, 16)] = zeros16
    pltpu.sync_copy(zden, den_sh.at[pl.ds(sid * RPT, RPT)])
    plsc.subcore_barrier()

    pltpu.sync_copy(src_hbm.at[wid], sv)
    pltpu.sync_copy(dst_hbm.at[wid], dv)

    ring = [(rb0, wv0, sg0, sr0, sw0), (rb1, wv1, sg1, sr1, sw1),
            (rb2, wv2, sg2, sr2, sw2), (rb3, wv3, sg3, sr3, sw3)]

    pltpu.async_copy(h_hbm.at[sv.at[0]], rb0, sg0)
    pltpu.async_copy(h_hbm.at[sv.at[1]], rb1, sg1)

    def _process1(j, q, have_old_w, have_prev_r, issue_next):
        rows, wv, sg, sr, sw = ring[q]
        qn = (q + 2) % 4
        rows_n, _, sgn, srn, _ = ring[qn]

        _when(have_old_w,
              lambda: pltpu.make_async_copy(wv, den_sh.at[dv.at[0]], sw).wait())

        @pl.loop(0, B // 16)
        def _(i):
            s16 = sv[j, pl.ds(i * 16, 16)]
            d16 = dv[j, pl.ds(i * 16, 16)]
            a_s = plsc.load_gather(as_t, [s16])
            a_d = plsc.load_gather(ad_t, [d16])
            mp = plsc.load_gather(mp_t, [d16])
            e = _leaky(a_s + a_d)
            wv[pl.ds(i * 16, 16)] = jnp.exp(e - mp)

        pltpu.make_async_copy(h_hbm.at[sv.at[j]], rows, sg).wait()

        @pl.loop(0, B // 16)
        def _(i):
            w16 = wv[pl.ds(i * 16, 16)]
            for l in range(16):
                w = w16[l]
                for c4 in range(H // 16):
                    r = i * 16 + l
                    rows[r, pl.ds(c4 * 16, 16)] = rows[r, pl.ds(c4 * 16, 16)] * w

        pltpu.async_copy(rows, acc_sh.at[dv.at[j]], sr, add=True)
        pltpu.async_copy(wv, den_sh.at[dv.at[j]], sw, add=True)

        def _issue():
            _when(have_prev_r,
                  lambda: pltpu.make_async_copy(
                      rows_n, acc_sh.at[dv.at[0]], srn).wait())
            pltpu.async_copy(h_hbm.at[sv.at[j + 2]], rows_n, sgn)
        _when(issue_next, _issue)

    @pl.loop(0, NSUB // 4)
    def _(k):
        _process1(4 * k + 0, 0, k > 0, k > 0, True)
        _process1(4 * k + 1, 1, k > 0, k > 0, True)
        _process1(4 * k + 2, 2, k > 0, True, True)
        _process1(4 * k + 3, 3, k > 0, True, k < NSUB // 4 - 1)

    _process1(NSUB - 1, 0, True, None, None)

    for q in (1, 2, 3, 0):
        rows, wv, _, sr, sw = ring[q]
        pltpu.make_async_copy(rows, acc_sh.at[dv.at[0]], sr).wait()
        pltpu.make_async_copy(wv, den_sh.at[dv.at[0]], sw).wait()

    plsc.subcore_barrier()
    pltpu.sync_copy(acc_sh.at[pl.ds(sid * RPT, RPT)],
                    acc_out.at[cid].at[pl.ds(sid * RPT, RPT)])
    pltpu.sync_copy(den_sh.at[pl.ds(sid * RPT, RPT)],
                    den_out.at[cid].at[pl.ds(sid * RPT, RPT)])


def _sc_layer2_body(src_hbm, dst_hbm, t_hbm,
                    n0_out, n1_out, den_out,
                    t_t, sv, dv,
                    wv0, p0v0, p1v0, wv1, p0v1, p1v1, zden,
                    n0_sh, n1_sh, den_sh, sem0, sem1):
    cid = lax.axis_index("c")
    sid = lax.axis_index("s")
    wid = cid * NS + sid

    pltpu.sync_copy(t_hbm, t_t)
    c0 = jnp.zeros((16,), jnp.int32)
    c1 = c0 + 1
    c2 = c0 + 2
    c3 = c0 + 3
    c4 = c0 + 4

    zeros16 = jnp.zeros((16,), _f32)

    @pl.loop(0, RPT // 16)
    def _(i):
        zden[pl.ds(i * 16, 16)] = zeros16
    pltpu.sync_copy(zden, n0_sh.at[pl.ds(sid * RPT, RPT)])
    pltpu.sync_copy(zden, n1_sh.at[pl.ds(sid * RPT, RPT)])
    pltpu.sync_copy(zden, den_sh.at[pl.ds(sid * RPT, RPT)])
    plsc.subcore_barrier()

    pltpu.sync_copy(src_hbm.at[wid], sv)
    pltpu.sync_copy(dst_hbm.at[wid], dv)

    def _process2(j, wv, p0v, p1v, sem, do_wait):
        def _wait_prev():
            pltpu.make_async_copy(p0v, n0_sh.at[dv.at[j]], sem).wait()
            pltpu.make_async_copy(p1v, n1_sh.at[dv.at[j]], sem).wait()
            pltpu.make_async_copy(wv, den_sh.at[dv.at[j]], sem).wait()
        _when(do_wait, _wait_prev)

        @pl.loop(0, B // 16)
        def _(i):
            s16 = sv[j, pl.ds(i * 16, 16)]
            d16 = dv[j, pl.ds(i * 16, 16)]
            a_s = plsc.load_gather(t_t, [s16, c0])
            a_d = plsc.load_gather(t_t, [d16, c1])
            mp = plsc.load_gather(t_t, [d16, c2])
            h0 = plsc.load_gather(t_t, [s16, c3])
            h1 = plsc.load_gather(t_t, [s16, c4])
            e = _leaky(a_s + a_d)
            w = jnp.exp(e - mp)
            sl = pl.ds(i * 16, 16)
            wv[sl] = w
            p0v[sl] = w * h0
            p1v[sl] = w * h1

        pltpu.async_copy(p0v, n0_sh.at[dv.at[j]], sem, add=True)
        pltpu.async_copy(p1v, n1_sh.at[dv.at[j]], sem, add=True)
        pltpu.async_copy(wv, den_sh.at[dv.at[j]], sem, add=True)

    @pl.loop(0, NSUB // 2)
    def _(k):
        _process2(2 * k, wv0, p0v0, p1v0, sem0, k > 0)
        _process2(2 * k + 1, wv1, p0v1, p1v1, sem1, k > 0)

    _process2(NSUB - 1, wv0, p0v0, p1v0, sem0, True)
    pltpu.make_async_copy(p0v0, n0_sh.at[dv.at[0]], sem0).wait()
    pltpu.make_async_copy(p1v0, n1_sh.at[dv.at[0]], sem0).wait()
    pltpu.make_async_copy(wv0, den_sh.at[dv.at[0]], sem0).wait()
    pltpu.make_async_copy(p0v1, n0_sh.at[dv.at[0]], sem1).wait()
    pltpu.make_async_copy(p1v1, n1_sh.at[dv.at[0]], sem1).wait()
    pltpu.make_async_copy(wv1, den_sh.at[dv.at[0]], sem1).wait()

    plsc.subcore_barrier()
    pltpu.sync_copy(n0_sh.at[pl.ds(sid * RPT, RPT)],
                    n0_out.at[cid].at[pl.ds(sid * RPT, RPT)])
    pltpu.sync_copy(n1_sh.at[pl.ds(sid * RPT, RPT)],
                    n1_out.at[cid].at[pl.ds(sid * RPT, RPT)])
    pltpu.sync_copy(den_sh.at[pl.ds(sid * RPT, RPT)],
                    den_out.at[cid].at[pl.ds(sid * RPT, RPT)])


def kernel(x, edge_index, W1, att_src1, att_dst1, b1, W2, att_src2, att_dst2, b2):
    src3d = edge_index[0].reshape(NW, NSUB, B)
    dst3d = edge_index[1].reshape(NW, NSUB, B)

    h1, as1, ad1, mp1 = pl.pallas_call(
        _tc_prologue_body,
        out_shape=[
            jax.ShapeDtypeStruct((N, H), _f32),
            jax.ShapeDtypeStruct((N, 1), _f32),
            jax.ShapeDtypeStruct((N, 1), _f32),
            jax.ShapeDtypeStruct((N, 1), _f32),
        ],
    )(x, W1, att_src1.reshape(1, H), att_dst1.reshape(1, H))

    mesh = plsc.VectorSubcoreMesh(core_axis_name="c", subcore_axis_name="s")
    cp = pltpu.CompilerParams()
    for _f, _v in (("needs_layout_passes", False), ("use_tc_tiling_on_sc", False)):
        if _f in pltpu.CompilerParams.__dataclass_fields__:
            cp = dataclasses.replace(cp, **{_f: _v})

    sc1 = functools.partial(
        pl.kernel,
        out_type=[
            jax.ShapeDtypeStruct((NC, NP, H), _f32),
            jax.ShapeDtypeStruct((NC, NP), _f32),
        ],
        mesh=mesh,
        compiler_params=cp,
        scratch_types=(
            [pltpu.VMEM((N,), _f32)] * 3 +
            [pltpu.VMEM((NSUB, B), jnp.int32)] * 2 +
            [pltpu.VMEM((B,), _f32)] * 4 +
            [pltpu.VMEM((B, H), _f32)] * 4 +
            [pltpu.VMEM((16, H), _f32),
             pltpu.VMEM((RPT,), _f32),
             pltpu.VMEM_SHARED((NP, H), _f32),
             pltpu.VMEM_SHARED((NP,), _f32)] +
            [pltpu.SemaphoreType.DMA] * 12
        ),
    )(_sc_layer1_body)
    accp, denp = sc1(src3d, dst3d,
                     as1.reshape(N), ad1.reshape(N), mp1.reshape(N), h1)

    t2 = pl.pallas_call(
        _tc_mid_body,
        out_shape=jax.ShapeDtypeStruct((NP, 8), _f32),
    )(accp, denp.reshape(NC, NP, 1), b1.reshape(1, H), W2,
      att_src2.reshape(1, C), att_dst2.reshape(1, C))

    sc2 = functools.partial(
        pl.kernel,
        out_type=[
            jax.ShapeDtypeStruct((NC, NP), _f32),
            jax.ShapeDtypeStruct((NC, NP), _f32),
            jax.ShapeDtypeStruct((NC, NP), _f32),
        ],
        mesh=mesh,
        compiler_params=cp,
        scratch_types=(
            [pltpu.VMEM((NP, 8), _f32)] +
            [pltpu.VMEM((NSUB, B), jnp.int32)] * 2 +
            [pltpu.VMEM((B,), _f32)] * 6 +
            [pltpu.VMEM((RPT,), _f32),
             pltpu.VMEM_SHARED((NP,), _f32),
             pltpu.VMEM_SHARED((NP,), _f32),
             pltpu.VMEM_SHARED((NP,), _f32)] +
            [pltpu.SemaphoreType.DMA] * 2
        ),
    )(_sc_layer2_body)
    n0, n1, den2 = sc2(src3d, dst3d, t2)

    out = pl.pallas_call(
        _tc_epilogue_body,
        out_shape=jax.ShapeDtypeStruct((NP, C), _f32),
    )(n0.reshape(NC, NP, 1), n1.reshape(NC, NP, 1), den2.reshape(NC, NP, 1),
      b2.reshape(1, C))

    return out[:N]


# speedup vs baseline: 1.2408x; 1.0254x over previous
import dataclasses
import functools

import jax
import jax.numpy as jnp
from jax import lax
from jax.experimental import pallas as pl
from jax.experimental.pallas import tpu as pltpu
from jax.experimental.pallas import tpu_sc as plsc

N = 10000
E = 320000
D = 128
H = 64
C = 2

NC = 2
NS = 16
NW = NC * NS
EW = E // NW
B = 80
NSUB = EW // B
NP = 10240
RPT = NP // NS

_f32 = jnp.float32


def _tc_prologue_body(x_ref, w_ref, asv_ref, adv_ref, h_out, t_out):
    h = jnp.dot(x_ref[...], w_ref[...], preferred_element_type=_f32)
    h_out[...] = h
    a_s = jnp.sum(h * asv_ref[...], axis=1, keepdims=True)
    a_d = jnp.sum(h * adv_ref[...], axis=1, keepdims=True)
    z = jnp.max(a_s) + a_d
    mp = jnp.where(z > 0, z, 0.2 * z)
    zero = jnp.zeros_like(a_s)
    t_out[...] = jnp.concatenate([a_s, a_d, mp, zero], axis=1)


def _tc_mid_body(accp_ref, denp_ref, b1_ref, w2_ref, asv_ref, adv_ref, t_out):
    acc = accp_ref[0] + accp_ref[1]
    den = denp_ref[0] + denp_ref[1]
    hr = jnp.maximum(acc / (den + 1e-16) + b1_ref[...], 0.0)
    h2 = jnp.dot(hr, w2_ref[...], preferred_element_type=_f32)
    a_s = jnp.sum(h2 * asv_ref[...], axis=1, keepdims=True)
    a_d = jnp.sum(h2 * adv_ref[...], axis=1, keepdims=True)
    z = jnp.max(a_s) + a_d
    mp = jnp.where(z > 0, z, 0.2 * z)
    zero = jnp.zeros_like(a_s)
    t_out[...] = jnp.concatenate(
        [a_s, a_d, mp, h2[:, 0:1], h2[:, 1:2], zero, zero, zero], axis=1)


def _tc_epilogue_body(n0_ref, n1_ref, d2_ref, b2_ref, out_ref):
    d = d2_ref[0] + d2_ref[1] + 1e-16
    o0 = (n0_ref[0] + n0_ref[1]) / d
    o1 = (n1_ref[0] + n1_ref[1]) / d
    out_ref[...] = jnp.concatenate([o0, o1], axis=1) + b2_ref[...]


def _leaky(e):
    return jnp.where(e > 0, e, 0.2 * e)


def _when(cond, fn):
    if cond is None:
        return
    if cond is True:
        fn()
        return

    @pl.when(cond)
    def _():
        fn()


def _sc_layer1_body(src_hbm, dst_hbm, t_hbm, h_hbm,
                    acc_out, den_out,
                    t_t, sv, dv,
                    wv0, wv1, wv2, wv3, rb0, rb1, rb2, rb3, zrow, zden,
                    acc_sh, den_sh,
                    sg0, sg1, sg2, sg3, sr0, sr1, sr2, sr3,
                    sw0, sw1, sw2, sw3):
    cid = lax.axis_index("c")
    sid = lax.axis_index("s")
    wid = cid * NS + sid

    pltpu.sync_copy(t_hbm, t_t)

    zeros16 = jnp.zeros((16,), _f32)

    @pl.loop(0, 16)
    def _(r):
        @pl.loop(0, H // 16)
        def _(c4):
            zrow[r, pl.ds(c4 * 16, 16)] = zeros16

    @pl.loop(0, RPT // 16)
    def _(i):
        pltpu.sync_copy(zrow, acc_sh.at[pl.ds(sid * RPT + i * 16, 16)])

    @pl.loop(0, RPT // 16)
    def _(i):
        zden[pl.ds(i * 16, 16)] = zeros16
    pltpu.sync_copy(zden, den_sh.at[pl.ds(sid * RPT, RPT)])
    plsc.subcore_barrier()

    pltpu.sync_copy(src_hbm.at[wid], sv)
    pltpu.sync_copy(dst_hbm.at[wid], dv)

    ring = [(rb0, wv0, sg0, sr0, sw0), (rb1, wv1, sg1, sr1, sw1),
            (rb2, wv2, sg2, sr2, sw2), (rb3, wv3, sg3, sr3, sw3)]

    pltpu.async_copy(h_hbm.at[sv.at[0]], rb0, sg0)
    pltpu.async_copy(h_hbm.at[sv.at[1]], rb1, sg1)

    def _process1(j, q, have_old_w, have_prev_r, issue_next):
        rows, wv, sg, sr, sw = ring[q]
        qn = (q + 2) % 4
        rows_n, _, sgn, srn, _ = ring[qn]

        _when(have_old_w,
              lambda: pltpu.make_async_copy(wv, den_sh.at[dv.at[0]], sw).wait())

        @pl.loop(0, B // 16)
        def _(i):
            s16 = sv[j, pl.ds(i * 16, 16)]
            d16 = dv[j, pl.ds(i * 16, 16)]
            s4 = s16 * 4
            d4 = d16 * 4
            a_s = plsc.load_gather(t_t, [s4])
            a_d = plsc.load_gather(t_t, [d4 + 1])
            mp = plsc.load_gather(t_t, [d4 + 2])
            e = _leaky(a_s + a_d)
            wv[pl.ds(i * 16, 16)] = jnp.exp(e - mp)

        pltpu.make_async_copy(h_hbm.at[sv.at[j]], rows, sg).wait()

        @pl.loop(0, B // 16)
        def _(i):
            w16 = wv[pl.ds(i * 16, 16)]
            for l in range(16):
                w = w16[l]
                for c4 in range(H // 16):
                    r = i * 16 + l
                    rows[r, pl.ds(c4 * 16, 16)] = rows[r, pl.ds(c4 * 16, 16)] * w

        pltpu.async_copy(rows, acc_sh.at[dv.at[j]], sr, add=True)
        pltpu.async_copy(wv, den_sh.at[dv.at[j]], sw, add=True)

        def _issue():
            _when(have_prev_r,
                  lambda: pltpu.make_async_copy(
                      rows_n, acc_sh.at[dv.at[0]], srn).wait())
            pltpu.async_copy(h_hbm.at[sv.at[j + 2]], rows_n, sgn)
        _when(issue_next, _issue)

    @pl.loop(0, NSUB // 4)
    def _(k):
        _process1(4 * k + 0, 0, k > 0, k > 0, True)
        _process1(4 * k + 1, 1, k > 0, k > 0, True)
        _process1(4 * k + 2, 2, k > 0, True, True)
        _process1(4 * k + 3, 3, k > 0, True, k < NSUB // 4 - 1)

    _process1(NSUB - 1, 0, True, None, None)

    for q in (1, 2, 3, 0):
        rows, wv, _, sr, sw = ring[q]
        pltpu.make_async_copy(rows, acc_sh.at[dv.at[0]], sr).wait()
        pltpu.make_async_copy(wv, den_sh.at[dv.at[0]], sw).wait()

    plsc.subcore_barrier()
    pltpu.sync_copy(acc_sh.at[pl.ds(sid * RPT, RPT)],
                    acc_out.at[cid].at[pl.ds(sid * RPT, RPT)])
    pltpu.sync_copy(den_sh.at[pl.ds(sid * RPT, RPT)],
                    den_out.at[cid].at[pl.ds(sid * RPT, RPT)])


def _sc_layer2_body(src_hbm, dst_hbm, t_hbm,
                    n0_out, n1_out, den_out,
                    t_t, sv, dv,
                    wv0, p0v0, p1v0, wv1, p0v1, p1v1, zden,
                    n0_sh, n1_sh, den_sh, sem0, sem1):
    cid = lax.axis_index("c")
    sid = lax.axis_index("s")
    wid = cid * NS + sid

    pltpu.sync_copy(t_hbm, t_t)
    c0 = jnp.zeros((16,), jnp.int32)
    c1 = c0 + 1
    c2 = c0 + 2
    c3 = c0 + 3
    c4 = c0 + 4

    zeros16 = jnp.zeros((16,), _f32)

    @pl.loop(0, RPT // 16)
    def _(i):
        zden[pl.ds(i * 16, 16)] = zeros16
    pltpu.sync_copy(zden, n0_sh.at[pl.ds(sid * RPT, RPT)])
    pltpu.sync_copy(zden, n1_sh.at[pl.ds(sid * RPT, RPT)])
    pltpu.sync_copy(zden, den_sh.at[pl.ds(sid * RPT, RPT)])
    plsc.subcore_barrier()

    pltpu.sync_copy(src_hbm.at[wid], sv)
    pltpu.sync_copy(dst_hbm.at[wid], dv)

    def _process2(j, wv, p0v, p1v, sem, do_wait):
        def _wait_prev():
            pltpu.make_async_copy(p0v, n0_sh.at[dv.at[j]], sem).wait()
            pltpu.make_async_copy(p1v, n1_sh.at[dv.at[j]], sem).wait()
            pltpu.make_async_copy(wv, den_sh.at[dv.at[j]], sem).wait()
        _when(do_wait, _wait_prev)

        @pl.loop(0, B // 16)
        def _(i):
            s16 = sv[j, pl.ds(i * 16, 16)]
            d16 = dv[j, pl.ds(i * 16, 16)]
            a_s = plsc.load_gather(t_t, [s16, c0])
            a_d = plsc.load_gather(t_t, [d16, c1])
            mp = plsc.load_gather(t_t, [d16, c2])
            h0 = plsc.load_gather(t_t, [s16, c3])
            h1 = plsc.load_gather(t_t, [s16, c4])
            e = _leaky(a_s + a_d)
            w = jnp.exp(e - mp)
            sl = pl.ds(i * 16, 16)
            wv[sl] = w
            p0v[sl] = w * h0
            p1v[sl] = w * h1

        pltpu.async_copy(p0v, n0_sh.at[dv.at[j]], sem, add=True)
        pltpu.async_copy(p1v, n1_sh.at[dv.at[j]], sem, add=True)
        pltpu.async_copy(wv, den_sh.at[dv.at[j]], sem, add=True)

    @pl.loop(0, NSUB // 2)
    def _(k):
        _process2(2 * k, wv0, p0v0, p1v0, sem0, k > 0)
        _process2(2 * k + 1, wv1, p0v1, p1v1, sem1, k > 0)

    _process2(NSUB - 1, wv0, p0v0, p1v0, sem0, True)
    pltpu.make_async_copy(p0v0, n0_sh.at[dv.at[0]], sem0).wait()
    pltpu.make_async_copy(p1v0, n1_sh.at[dv.at[0]], sem0).wait()
    pltpu.make_async_copy(wv0, den_sh.at[dv.at[0]], sem0).wait()
    pltpu.make_async_copy(p0v1, n0_sh.at[dv.at[0]], sem1).wait()
    pltpu.make_async_copy(p1v1, n1_sh.at[dv.at[0]], sem1).wait()
    pltpu.make_async_copy(wv1, den_sh.at[dv.at[0]], sem1).wait()

    plsc.subcore_barrier()
    pltpu.sync_copy(n0_sh.at[pl.ds(sid * RPT, RPT)],
                    n0_out.at[cid].at[pl.ds(sid * RPT, RPT)])
    pltpu.sync_copy(n1_sh.at[pl.ds(sid * RPT, RPT)],
                    n1_out.at[cid].at[pl.ds(sid * RPT, RPT)])
    pltpu.sync_copy(den_sh.at[pl.ds(sid * RPT, RPT)],
                    den_out.at[cid].at[pl.ds(sid * RPT, RPT)])


def kernel(x, edge_index, W1, att_src1, att_dst1, b1, W2, att_src2, att_dst2, b2):
    src3d = edge_index[0].reshape(NW, NSUB, B)
    dst3d = edge_index[1].reshape(NW, NSUB, B)

    h1, t1 = pl.pallas_call(
        _tc_prologue_body,
        out_shape=[
            jax.ShapeDtypeStruct((N, H), _f32),
            jax.ShapeDtypeStruct((N, 4), _f32),
        ],
    )(x, W1, att_src1.reshape(1, H), att_dst1.reshape(1, H))

    mesh = plsc.VectorSubcoreMesh(core_axis_name="c", subcore_axis_name="s")
    cp = pltpu.CompilerParams()
    for _f, _v in (("needs_layout_passes", False), ("use_tc_tiling_on_sc", False)):
        if _f in pltpu.CompilerParams.__dataclass_fields__:
            cp = dataclasses.replace(cp, **{_f: _v})

    sc1 = functools.partial(
        pl.kernel,
        out_type=[
            jax.ShapeDtypeStruct((NC, NP, H), _f32),
            jax.ShapeDtypeStruct((NC, NP), _f32),
        ],
        mesh=mesh,
        compiler_params=cp,
        scratch_types=(
            [pltpu.VMEM((N * 4,), _f32)] +
            [pltpu.VMEM((NSUB, B), jnp.int32)] * 2 +
            [pltpu.VMEM((B,), _f32)] * 4 +
            [pltpu.VMEM((B, H), _f32)] * 4 +
            [pltpu.VMEM((16, H), _f32),
             pltpu.VMEM((RPT,), _f32),
             pltpu.VMEM_SHARED((NP, H), _f32),
             pltpu.VMEM_SHARED((NP,), _f32)] +
            [pltpu.SemaphoreType.DMA] * 12
        ),
    )(_sc_layer1_body)
    accp, denp = sc1(src3d, dst3d, t1.reshape(N * 4), h1)

    t2 = pl.pallas_call(
        _tc_mid_body,
        out_shape=jax.ShapeDtypeStruct((NP, 8), _f32),
    )(accp, denp.reshape(NC, NP, 1), b1.reshape(1, H), W2,
      att_src2.reshape(1, C), att_dst2.reshape(1, C))

    sc2 = functools.partial(
        pl.kernel,
        out_type=[
            jax.ShapeDtypeStruct((NC, NP), _f32),
            jax.ShapeDtypeStruct((NC, NP), _f32),
            jax.ShapeDtypeStruct((NC, NP), _f32),
        ],
        mesh=mesh,
        compiler_params=cp,
        scratch_types=(
            [pltpu.VMEM((NP, 8), _f32)] +
            [pltpu.VMEM((NSUB, B), jnp.int32)] * 2 +
            [pltpu.VMEM((B,), _f32)] * 6 +
            [pltpu.VMEM((RPT,), _f32),
             pltpu.VMEM_SHARED((NP,), _f32),
             pltpu.VMEM_SHARED((NP,), _f32),
             pltpu.VMEM_SHARED((NP,), _f32)] +
            [pltpu.SemaphoreType.DMA] * 2
        ),
    )(_sc_layer2_body)
    n0, n1, den2 = sc2(src3d, dst3d, t2)

    out = pl.pallas_call(
        _tc_epilogue_body,
        out_shape=jax.ShapeDtypeStruct((NP, C), _f32),
    )(n0.reshape(NC, NP, 1), n1.reshape(NC, NP, 1), den2.reshape(NC, NP, 1),
      b2.reshape(1, C))

    return out[:N]
